# Initial kernel scaffold; baseline (speedup 1.0000x reference)
import jax
import jax.numpy as jnp
from jax.experimental import pallas as pl


def kernel(x, edge_index, batch, W1, b1, W2, b2, W3, b3, Wl1, bl1, Wl2, bl2):
    raise NotImplementedError("write your pallas kernel here")


# speedup vs baseline: 8.5266x; 8.5266x over previous
import functools

import jax
import jax.numpy as jnp
from jax import lax
from jax.experimental import pallas as pl
from jax.experimental.pallas import tpu as pltpu
from jax.experimental.pallas import tpu_sc as plsc

N = 10000
E = 320000
F = 128
C = 10
G = 64

NC, NS = 2, 16
NW = NC * NS
NPAD = 10112
SUBROWS = NPAD // NS
CH = 128
NCHUNK = 80
EPW = NCHUNK * CH
EP = NW * EPW

_HIGH = jax.lax.Precision.HIGHEST


@functools.cache
def _mesh():
    return plsc.VectorSubcoreMesh(core_axis_name="c", subcore_axis_name="s",
                                  num_cores=NC, num_subcores=NS)


def _deg_body(dstv_hbm, out_hbm, dst_v, hist_v):
    c = lax.axis_index("c")
    s = lax.axis_index("s")
    wid = s * NC + c
    pltpu.sync_copy(dstv_hbm.at[wid], dst_v)

    zeros16 = jnp.zeros((16,), jnp.float32)
    def _zero(i, carry):
        hist_v[pl.ds(i * 16, 16)] = zeros16
        return carry
    lax.fori_loop(0, NPAD // 16, _zero, 0)

    ones16 = jnp.ones((16,), jnp.float32)
    def _acc(i, carry):
        idx = dst_v[i, :]
        plsc.addupdate_scatter(hist_v, [idx], ones16)
        return carry
    lax.fori_loop(0, EPW // 16, _acc, 0)

    pltpu.sync_copy(hist_v, out_hbm.at[wid])


@functools.cache
def _deg_kernel():
    return pl.kernel(
        _deg_body,
        out_type=jax.ShapeDtypeStruct((NW, NPAD), jnp.float32),
        mesh=_mesh(),
        scratch_types=[
            pltpu.VMEM((EPW // 16, 16), jnp.int32),
            pltpu.VMEM((NPAD,), jnp.float32),
        ],
        compiler_params=pltpu.CompilerParams(needs_layout_passes=False),
    )


def _mp_body(y_hbm, sdv_hbm, zrows_hbm, out_hbm,
             iwa, iwb, bufa, bufb, acc_sh, isa, isb, rsa, rsb):
    c = lax.axis_index("c")
    s = lax.axis_index("s")
    wid = s * NC + c
    pltpu.sync_copy(zrows_hbm, acc_sh.at[pl.ds(s * SUBROWS, SUBROWS)])

    pltpu.async_copy(sdv_hbm.at[wid, 0], iwa, isa)
    pltpu.async_copy(sdv_hbm.at[wid, 1], iwb, isb)
    plsc.subcore_barrier()
    pltpu.make_async_copy(sdv_hbm.at[wid, 0], iwa, isa).wait()
    pltpu.async_copy(y_hbm.at[iwa.at[0]], bufa, rsa)
    pltpu.make_async_copy(sdv_hbm.at[wid, 1], iwb, isb).wait()
    pltpu.async_copy(y_hbm.at[iwb.at[0]], bufb, rsb)

    def _pair(g, carry):
        j0 = 2 * g
        pltpu.make_async_copy(y_hbm.at[iwa.at[0]], bufa, rsa).wait()
        pltpu.sync_copy(bufa, acc_sh.at[iwa.at[1]], add=True)
        @pl.when(g < NCHUNK // 2 - 1)
        def _():
            pltpu.async_copy(sdv_hbm.at[wid, j0 + 2], iwa, isa)
            pltpu.make_async_copy(sdv_hbm.at[wid, j0 + 2], iwa, isa).wait()
            pltpu.async_copy(y_hbm.at[iwa.at[0]], bufa, rsa)
        pltpu.make_async_copy(y_hbm.at[iwb.at[0]], bufb, rsb).wait()
        pltpu.sync_copy(bufb, acc_sh.at[iwb.at[1]], add=True)
        @pl.when(g < NCHUNK // 2 - 1)
        def _():
            pltpu.async_copy(sdv_hbm.at[wid, j0 + 3], iwb, isb)
            pltpu.make_async_copy(sdv_hbm.at[wid, j0 + 3], iwb, isb).wait()
            pltpu.async_copy(y_hbm.at[iwb.at[0]], bufb, rsb)
        return carry
    lax.fori_loop(0, NCHUNK // 2, _pair, 0)

    plsc.subcore_barrier()
    pltpu.sync_copy(acc_sh.at[pl.ds(s * SUBROWS, SUBROWS)],
                    out_hbm.at[c, pl.ds(s * SUBROWS, SUBROWS)])


@functools.cache
def _mp_kernel():
    return pl.kernel(
        _mp_body,
        out_type=jax.ShapeDtypeStruct((NC, NPAD, F), jnp.float32),
        mesh=_mesh(),
        scratch_types=[
            pltpu.VMEM((2, CH), jnp.int32),
            pltpu.VMEM((2, CH), jnp.int32),
            pltpu.VMEM((CH, F), jnp.float32),
            pltpu.VMEM((CH, F), jnp.float32),
            pltpu.VMEM_SHARED((NPAD, F), jnp.float32),
            pltpu.SemaphoreType.DMA,
            pltpu.SemaphoreType.DMA,
            pltpu.SemaphoreType.DMA,
            pltpu.SemaphoreType.DMA,
        ],
        compiler_params=pltpu.CompilerParams(needs_layout_passes=False),
    )


def _prep_body(degp_ref, dinv_ref):
    deg = jnp.sum(degp_ref[...], axis=0, keepdims=True) + 1.0
    dinv_ref[...] = lax.rsqrt(deg)


def _l1_body(x_ref, W_ref, dinv_ref, y_ref):
    xw = jnp.dot(x_ref[...], W_ref[...],
                 preferred_element_type=jnp.float32, precision=_HIGH)
    y_ref[...] = xw * dinv_ref[...]


def _layer_body(acc_ref, yprev_ref, dinv_ref, b_ref, W_ref, y_ref):
    pre = (acc_ref[0] + acc_ref[1] + yprev_ref[...]) * dinv_ref[...] + b_ref[...]
    h = jnp.maximum(pre, 0.0)
    y_ref[...] = jnp.dot(h, W_ref[...],
                         preferred_element_type=jnp.float32,
                         precision=_HIGH) * dinv_ref[...]


def _final_body(acc_ref, y3_ref, dinv_ref, b3_ref, batch_ref,
                Wl1_ref, bl1_ref, Wl2_ref, bl2_ref, out_ref):
    out3 = (acc_ref[0] + acc_ref[1] + y3_ref[...]) * dinv_ref[...] + b3_ref[...]
    gid = lax.broadcasted_iota(jnp.int32, (G, NPAD), 0)
    maskT = (batch_ref[...] == gid).astype(jnp.float32)
    psum = jnp.dot(maskT, out3,
                   preferred_element_type=jnp.float32, precision=_HIGH)
    counts = jnp.sum(maskT, axis=1, keepdims=True)
    pooled = psum / jnp.maximum(counts, 1.0)
    z = jnp.maximum(
        jnp.dot(pooled, Wl1_ref[...],
                preferred_element_type=jnp.float32, precision=_HIGH)
        + bl1_ref[...], 0.0)
    out_ref[...] = jnp.dot(z, Wl2_ref[...],
                           preferred_element_type=jnp.float32,
                           precision=_HIGH) + bl2_ref[...]


def _tc_call(body, out_shape):
    return pl.pallas_call(body, out_shape=out_shape)


def kernel(x, edge_index, batch, W1, b1, W2, b2, W3, b3, Wl1, bl1, Wl2, bl2):
    f32 = jnp.float32
    src = jnp.concatenate([edge_index[0], jnp.zeros((EP - E,), jnp.int32)])
    dst = jnp.concatenate([edge_index[1],
                           jnp.full((EP - E,), N, jnp.int32)])
    srcv = src.reshape(NW, NCHUNK, CH)
    dstv = dst.reshape(NW, NCHUNK, CH)
    sdv = jnp.stack([srcv, dstv], axis=2)
    x_p = jnp.concatenate([x, jnp.zeros((NPAD - N, F), f32)], axis=0)
    batch_row = jnp.concatenate([batch,
                                 jnp.full((NPAD - N,), G, jnp.int32)]
                                ).reshape(1, NPAD)
    zrows = jnp.zeros((SUBROWS, F), f32)

    deg_part = _deg_kernel()(dstv.reshape(NW, EPW // 16, 16))
    dinv_row = _tc_call(_prep_body,
                        jax.ShapeDtypeStruct((1, NPAD), f32))(deg_part)
    dinv = dinv_row.reshape(NPAD, 1)

    y_shape = jax.ShapeDtypeStruct((NPAD, F), f32)
    y1 = _tc_call(_l1_body, y_shape)(x_p, W1, dinv)
    acc1 = _mp_kernel()(y1, sdv, zrows)
    y2 = _tc_call(_layer_body, y_shape)(acc1, y1, dinv, b1.reshape(1, F), W2)
    acc2 = _mp_kernel()(y2, sdv, zrows)
    y3 = _tc_call(_layer_body, y_shape)(acc2, y2, dinv, b2.reshape(1, F), W3)
    acc3 = _mp_kernel()(y3, sdv, zrows)

    out = _tc_call(_final_body, jax.ShapeDtypeStruct((G, C), f32))(
        acc3, y3, dinv, b3.reshape(1, F), batch_row,
        Wl1, bl1.reshape(1, F), Wl2, bl2.reshape(1, C))
    return out


# speedup vs baseline: 25.4881x; 2.9892x over previous
import functools

import jax
import jax.numpy as jnp
from jax import lax
from jax.experimental import pallas as pl
from jax.experimental.pallas import tpu as pltpu
from jax.experimental.pallas import tpu_sc as plsc

N = 10000
E = 320000
F = 128
C = 10
G = 64

NC, NS = 2, 16
NW = NC * NS
NPAD = 10112
SUBROWS = NPAD // NS
CH = 128
NCHUNK = 80
EPW = NCHUNK * CH
EP = NW * EPW

_HIGH = jax.lax.Precision.HIGHEST


@functools.cache
def _mesh():
    return plsc.VectorSubcoreMesh(core_axis_name="c", subcore_axis_name="s",
                                  num_cores=NC, num_subcores=NS)


def _deg_body(dstv_hbm, out_hbm, dst_v, hist_v):
    c = lax.axis_index("c")
    s = lax.axis_index("s")
    wid = s * NC + c
    pltpu.sync_copy(dstv_hbm.at[wid], dst_v)

    zeros16 = jnp.zeros((16,), jnp.float32)
    def _zero(i, carry):
        hist_v[pl.ds(i * 16, 16)] = zeros16
        return carry
    lax.fori_loop(0, NPAD // 16, _zero, 0)

    ones16 = jnp.ones((16,), jnp.float32)
    def _acc(i, carry):
        idx = dst_v[i, :]
        plsc.addupdate_scatter(hist_v, [idx], ones16)
        return carry
    lax.fori_loop(0, EPW // 16, _acc, 0)

    pltpu.sync_copy(hist_v, out_hbm.at[wid])


@functools.cache
def _deg_kernel():
    return pl.kernel(
        _deg_body,
        out_type=jax.ShapeDtypeStruct((NW, NPAD), jnp.float32),
        mesh=_mesh(),
        scratch_types=[
            pltpu.VMEM((EPW // 16, 16), jnp.int32),
            pltpu.VMEM((NPAD,), jnp.float32),
        ],
        compiler_params=pltpu.CompilerParams(needs_layout_passes=False),
    )


def _mp_body(y_hbm, sdv_hbm, zrows_hbm, out_hbm,
             iwa, iwb, bufa, bufb, acc_sh, isa, isb, rsa, rsb):
    c = lax.axis_index("c")
    s = lax.axis_index("s")
    wid = s * NC + c
    pltpu.sync_copy(zrows_hbm, acc_sh.at[pl.ds(s * SUBROWS, SUBROWS)])

    pltpu.async_copy(sdv_hbm.at[wid, 0], iwa, isa)
    pltpu.async_copy(sdv_hbm.at[wid, 1], iwb, isb)
    plsc.subcore_barrier()
    pltpu.make_async_copy(sdv_hbm.at[wid, 0], iwa, isa).wait()
    pltpu.async_copy(y_hbm.at[iwa.at[0]], bufa, rsa)
    pltpu.make_async_copy(sdv_hbm.at[wid, 1], iwb, isb).wait()
    pltpu.async_copy(y_hbm.at[iwb.at[0]], bufb, rsb)

    def _pair(g, carry):
        j0 = 2 * g
        pltpu.make_async_copy(y_hbm.at[iwa.at[0]], bufa, rsa).wait()
        pltpu.sync_copy(bufa, acc_sh.at[iwa.at[1]], add=True)
        @pl.when(g < NCHUNK // 2 - 1)
        def _():
            pltpu.async_copy(sdv_hbm.at[wid, j0 + 2], iwa, isa)
            pltpu.make_async_copy(sdv_hbm.at[wid, j0 + 2], iwa, isa).wait()
            pltpu.async_copy(y_hbm.at[iwa.at[0]], bufa, rsa)
        pltpu.make_async_copy(y_hbm.at[iwb.at[0]], bufb, rsb).wait()
        pltpu.sync_copy(bufb, acc_sh.at[iwb.at[1]], add=True)
        @pl.when(g < NCHUNK // 2 - 1)
        def _():
            pltpu.async_copy(sdv_hbm.at[wid, j0 + 3], iwb, isb)
            pltpu.make_async_copy(sdv_hbm.at[wid, j0 + 3], iwb, isb).wait()
            pltpu.async_copy(y_hbm.at[iwb.at[0]], bufb, rsb)
        return carry
    lax.fori_loop(0, NCHUNK // 2, _pair, 0)

    plsc.subcore_barrier()
    pltpu.sync_copy(acc_sh.at[pl.ds(s * SUBROWS, SUBROWS)],
                    out_hbm.at[c, pl.ds(s * SUBROWS, SUBROWS)])


@functools.cache
def _mp_kernel():
    return pl.kernel(
        _mp_body,
        out_type=jax.ShapeDtypeStruct((NC, NPAD, F), jnp.float32),
        mesh=_mesh(),
        scratch_types=[
            pltpu.VMEM((2, CH), jnp.int32),
            pltpu.VMEM((2, CH), jnp.int32),
            pltpu.VMEM((CH, F), jnp.float32),
            pltpu.VMEM((CH, F), jnp.float32),
            pltpu.VMEM_SHARED((NPAD, F), jnp.float32),
            pltpu.SemaphoreType.DMA,
            pltpu.SemaphoreType.DMA,
            pltpu.SemaphoreType.DMA,
            pltpu.SemaphoreType.DMA,
        ],
        compiler_params=pltpu.CompilerParams(needs_layout_passes=False),
    )


def _prep_body(degp_ref, dinv_ref):
    deg = jnp.sum(degp_ref[...], axis=0, keepdims=True) + 1.0
    dinv_ref[...] = lax.rsqrt(deg)


def _l1_body(x_ref, W_ref, dinv_ref, y_ref):
    xw = jnp.dot(x_ref[...], W_ref[...],
                 preferred_element_type=jnp.float32, precision=_HIGH)
    y_ref[...] = xw * dinv_ref[...]


def _layer_body(acc_ref, yprev_ref, dinv_ref, b_ref, W_ref, y_ref):
    pre = (acc_ref[0] + acc_ref[1] + yprev_ref[...]) * dinv_ref[...] + b_ref[...]
    h = jnp.maximum(pre, 0.0)
    y_ref[...] = jnp.dot(h, W_ref[...],
                         preferred_element_type=jnp.float32,
                         precision=_HIGH) * dinv_ref[...]


def _final_body(acc_ref, y3_ref, dinv_ref, b3_ref, batch_ref,
                Wl1_ref, bl1_ref, Wl2_ref, bl2_ref, out_ref):
    out3 = (acc_ref[0] + acc_ref[1] + y3_ref[...]) * dinv_ref[...] + b3_ref[...]
    gid = lax.broadcasted_iota(jnp.int32, (G, NPAD), 0)
    maskT = (batch_ref[...] == gid).astype(jnp.float32)
    psum = jnp.dot(maskT, out3,
                   preferred_element_type=jnp.float32, precision=_HIGH)
    counts = jnp.sum(maskT, axis=1, keepdims=True)
    pooled = psum / jnp.maximum(counts, 1.0)
    z = jnp.maximum(
        jnp.dot(pooled, Wl1_ref[...],
                preferred_element_type=jnp.float32, precision=_HIGH)
        + bl1_ref[...], 0.0)
    out_ref[...] = jnp.dot(z, Wl2_ref[...],
                           preferred_element_type=jnp.float32,
                           precision=_HIGH) + bl2_ref[...]


def _tc_call(body, out_shape):
    return pl.pallas_call(body, out_shape=out_shape)


def kernel(x, edge_index, batch, W1, b1, W2, b2, W3, b3, Wl1, bl1, Wl2, bl2):
    f32 = jnp.float32
    pads = jnp.arange(EP - E, dtype=jnp.int32)
    src = jnp.concatenate([edge_index[0], pads % N])
    dst = jnp.concatenate([edge_index[1], N + pads % (NPAD - N)])
    srcv = src.reshape(NW, NCHUNK, CH)
    dstv = dst.reshape(NW, NCHUNK, CH)
    sdv = jnp.stack([srcv, dstv], axis=2)
    x_p = jnp.concatenate([x, jnp.zeros((NPAD - N, F), f32)], axis=0)
    batch_row = jnp.concatenate([batch,
                                 jnp.full((NPAD - N,), G, jnp.int32)]
                                ).reshape(1, NPAD)
    zrows = jnp.zeros((SUBROWS, F), f32)

    deg_part = _deg_kernel()(dstv.reshape(NW, EPW // 16, 16))
    dinv_row = _tc_call(_prep_body,
                        jax.ShapeDtypeStruct((1, NPAD), f32))(deg_part)
    dinv = dinv_row.reshape(NPAD, 1)

    y_shape = jax.ShapeDtypeStruct((NPAD, F), f32)
    y1 = _tc_call(_l1_body, y_shape)(x_p, W1, dinv)
    acc1 = _mp_kernel()(y1, sdv, zrows)
    y2 = _tc_call(_layer_body, y_shape)(acc1, y1, dinv, b1.reshape(1, F), W2)
    acc2 = _mp_kernel()(y2, sdv, zrows)
    y3 = _tc_call(_layer_body, y_shape)(acc2, y2, dinv, b2.reshape(1, F), W3)
    acc3 = _mp_kernel()(y3, sdv, zrows)

    out = _tc_call(_final_body, jax.ShapeDtypeStruct((G, C), f32))(
        acc3, y3, dinv, b3.reshape(1, F), batch_row,
        Wl1, bl1.reshape(1, F), Wl2, bl2.reshape(1, C))
    return out


# speedup vs baseline: 26.8766x; 1.0545x over previous
import functools

import jax
import jax.numpy as jnp
from jax import lax
from jax.experimental import pallas as pl
from jax.experimental.pallas import tpu as pltpu
from jax.experimental.pallas import tpu_sc as plsc

N = 10000
E = 320000
F = 128
C = 10
G = 64

NC, NS = 2, 16
NW = NC * NS
NPAD = 10112
SUBROWS = NPAD // NS
CH = 128
NCHUNK = 81
EPW = NCHUNK * CH
EP = NW * EPW

_HIGH = jax.lax.Precision.HIGHEST


@functools.cache
def _mesh():
    return plsc.VectorSubcoreMesh(core_axis_name="c", subcore_axis_name="s",
                                  num_cores=NC, num_subcores=NS)


def _deg_body(dstv_hbm, out_hbm, dst_v, hist_v):
    c = lax.axis_index("c")
    s = lax.axis_index("s")
    wid = s * NC + c
    pltpu.sync_copy(dstv_hbm.at[wid], dst_v)

    zeros16 = jnp.zeros((16,), jnp.float32)
    def _zero(i, carry):
        hist_v[pl.ds(i * 16, 16)] = zeros16
        return carry
    lax.fori_loop(0, NPAD // 16, _zero, 0)

    ones16 = jnp.ones((16,), jnp.float32)
    def _acc(i, carry):
        idx = dst_v[i, :]
        plsc.addupdate_scatter(hist_v, [idx], ones16)
        return carry
    lax.fori_loop(0, EPW // 16, _acc, 0)

    pltpu.sync_copy(hist_v, out_hbm.at[wid])


@functools.cache
def _deg_kernel():
    return pl.kernel(
        _deg_body,
        out_type=jax.ShapeDtypeStruct((NW, NPAD), jnp.float32),
        mesh=_mesh(),
        scratch_types=[
            pltpu.VMEM((EPW // 16, 16), jnp.int32),
            pltpu.VMEM((NPAD,), jnp.float32),
        ],
        compiler_params=pltpu.CompilerParams(needs_layout_passes=False),
    )


def _mp_body(y_hbm, sdv_hbm, zrows_hbm, out_hbm,
             iw0, iw1, iw2, buf0, buf1, buf2, acc_sh,
             rs0, rs1, rs2, ss0, ss1, ss2):
    c = lax.axis_index("c")
    s = lax.axis_index("s")
    wid = s * NC + c
    iws = (iw0, iw1, iw2)
    bufs = (buf0, buf1, buf2)
    rss = (rs0, rs1, rs2)
    sss = (ss0, ss1, ss2)

    pltpu.sync_copy(zrows_hbm, acc_sh.at[pl.ds(s * SUBROWS, SUBROWS)])
    for t in range(3):
        pltpu.sync_copy(sdv_hbm.at[wid, t], iws[t])
        pltpu.async_copy(y_hbm.at[iws[t].at[0]], bufs[t], rss[t])
    plsc.subcore_barrier()

    def _triplet(g, carry):
        j0 = 3 * g
        for t in range(3):
            pltpu.make_async_copy(y_hbm.at[iws[t].at[0]], bufs[t],
                                  rss[t]).wait()
            pltpu.async_copy(bufs[t], acc_sh.at[iws[t].at[1]], sss[t],
                             add=True)
        @pl.when(g < NCHUNK // 3 - 1)
        def _():
            for t in range(3):
                pltpu.make_async_copy(bufs[t], acc_sh.at[iws[t].at[1]],
                                      sss[t]).wait()
                pltpu.sync_copy(sdv_hbm.at[wid, j0 + 3 + t], iws[t])
                pltpu.async_copy(y_hbm.at[iws[t].at[0]], bufs[t], rss[t])
        return carry
    lax.fori_loop(0, NCHUNK // 3, _triplet, 0)

    for t in range(3):
        pltpu.make_async_copy(bufs[t], acc_sh.at[iws[t].at[1]],
                              sss[t]).wait()
    plsc.subcore_barrier()
    pltpu.sync_copy(acc_sh.at[pl.ds(s * SUBROWS, SUBROWS)],
                    out_hbm.at[c, pl.ds(s * SUBROWS, SUBROWS)])


@functools.cache
def _mp_kernel():
    return pl.kernel(
        _mp_body,
        out_type=jax.ShapeDtypeStruct((NC, NPAD, F), jnp.float32),
        mesh=_mesh(),
        scratch_types=[
            pltpu.VMEM((2, CH), jnp.int32),
            pltpu.VMEM((2, CH), jnp.int32),
            pltpu.VMEM((2, CH), jnp.int32),
            pltpu.VMEM((CH, F), jnp.float32),
            pltpu.VMEM((CH, F), jnp.float32),
            pltpu.VMEM((CH, F), jnp.float32),
            pltpu.VMEM_SHARED((NPAD, F), jnp.float32),
            pltpu.SemaphoreType.DMA,
            pltpu.SemaphoreType.DMA,
            pltpu.SemaphoreType.DMA,
            pltpu.SemaphoreType.DMA,
            pltpu.SemaphoreType.DMA,
            pltpu.SemaphoreType.DMA,
        ],
        compiler_params=pltpu.CompilerParams(needs_layout_passes=False),
    )


def _prep_body(degp_ref, dinv_ref):
    deg = jnp.sum(degp_ref[...], axis=0, keepdims=True) + 1.0
    r = lax.rsqrt(deg)
    dinv_ref[...] = r * (1.5 - 0.5 * deg * r * r)


def _l1_body(x_ref, W_ref, dinv_ref, y_ref):
    xw = jnp.dot(x_ref[...], W_ref[...],
                 preferred_element_type=jnp.float32)
    y_ref[...] = xw * dinv_ref[...]


def _layer_body(acc_ref, yprev_ref, dinv_ref, b_ref, W_ref, y_ref):
    pre = (acc_ref[0] + acc_ref[1] + yprev_ref[...]) * dinv_ref[...] + b_ref[...]
    h = jnp.maximum(pre, 0.0)
    y_ref[...] = jnp.dot(h, W_ref[...],
                         preferred_element_type=jnp.float32) * dinv_ref[...]


def _final_body(acc_ref, y3_ref, dinv_ref, b3_ref, batch_ref,
                Wl1_ref, bl1_ref, Wl2_ref, bl2_ref, out_ref):
    out3 = (acc_ref[0] + acc_ref[1] + y3_ref[...]) * dinv_ref[...] + b3_ref[...]
    gid = lax.broadcasted_iota(jnp.int32, (G, NPAD), 0)
    maskT = (batch_ref[...] == gid).astype(jnp.float32)
    psum = jnp.dot(maskT, out3,
                   preferred_element_type=jnp.float32, precision=_HIGH)
    counts = jnp.sum(maskT, axis=1, keepdims=True)
    pooled = psum / jnp.maximum(counts, 1.0)
    z = jnp.maximum(
        jnp.dot(pooled, Wl1_ref[...],
                preferred_element_type=jnp.float32)
        + bl1_ref[...], 0.0)
    out_ref[...] = jnp.dot(z, Wl2_ref[...],
                           preferred_element_type=jnp.float32) + bl2_ref[...]


def _tc_call(body, out_shape):
    return pl.pallas_call(body, out_shape=out_shape)


def kernel(x, edge_index, batch, W1, b1, W2, b2, W3, b3, Wl1, bl1, Wl2, bl2):
    f32 = jnp.float32
    pads = jnp.arange(EP - E, dtype=jnp.int32)
    src = jnp.concatenate([edge_index[0], pads % N])
    dst = jnp.concatenate([edge_index[1], N + pads % (NPAD - N)])
    srcv = src.reshape(NW, NCHUNK, CH)
    dstv = dst.reshape(NW, NCHUNK, CH)
    sdv = jnp.stack([srcv, dstv], axis=2)
    x_p = jnp.concatenate([x, jnp.zeros((NPAD - N, F), f32)], axis=0)
    batch_row = jnp.concatenate([batch,
                                 jnp.full((NPAD - N,), G, jnp.int32)]
                                ).reshape(1, NPAD)
    zrows = jnp.zeros((SUBROWS, F), f32)

    deg_part = _deg_kernel()(dstv.reshape(NW, EPW // 16, 16))
    dinv_row = _tc_call(_prep_body,
                        jax.ShapeDtypeStruct((1, NPAD), f32))(deg_part)
    dinv = dinv_row.reshape(NPAD, 1)

    y_shape = jax.ShapeDtypeStruct((NPAD, F), f32)
    y1 = _tc_call(_l1_body, y_shape)(x_p, W1, dinv)
    acc1 = _mp_kernel()(y1, sdv, zrows)
    y2 = _tc_call(_layer_body, y_shape)(acc1, y1, dinv, b1.reshape(1, F), W2)
    acc2 = _mp_kernel()(y2, sdv, zrows)
    y3 = _tc_call(_layer_body, y_shape)(acc2, y2, dinv, b2.reshape(1, F), W3)
    acc3 = _mp_kernel()(y3, sdv, zrows)

    out = _tc_call(_final_body, jax.ShapeDtypeStruct((G, C), f32))(
        acc3, y3, dinv, b3.reshape(1, F), batch_row,
        Wl1, bl1.reshape(1, F), Wl2, bl2.reshape(1, C))
    return out


# speedup vs baseline: 27.5067x; 1.0234x over previous
import functools

import jax
import jax.numpy as jnp
from jax import lax
from jax.experimental import pallas as pl
from jax.experimental.pallas import tpu as pltpu
from jax.experimental.pallas import tpu_sc as plsc

N = 10000
E = 320000
F = 128
C = 10
G = 64

NC, NS = 2, 16
NW = NC * NS
NPAD = 10112
SUBROWS = NPAD // NS
CH = 128
NCHUNK = 81
EPW = NCHUNK * CH
EP = NW * EPW

_HIGH = jax.lax.Precision.HIGHEST


@functools.cache
def _mesh():
    return plsc.VectorSubcoreMesh(core_axis_name="c", subcore_axis_name="s",
                                  num_cores=NC, num_subcores=NS)


def _deg_body(dstv_hbm, out_hbm, dst_v, hist_v):
    c = lax.axis_index("c")
    s = lax.axis_index("s")
    wid = s * NC + c
    pltpu.sync_copy(dstv_hbm.at[wid], dst_v)

    zeros16 = jnp.zeros((16,), jnp.float32)
    def _zero(i, carry):
        hist_v[pl.ds(i * 16, 16)] = zeros16
        return carry
    lax.fori_loop(0, NPAD // 16, _zero, 0)

    ones16 = jnp.ones((16,), jnp.float32)
    def _acc(j, carry):
        def _acc16(k, carry2):
            idx = dst_v[j, pl.ds(k * 16, 16)]
            plsc.addupdate_scatter(hist_v, [idx], ones16)
            return carry2
        lax.fori_loop(0, CH // 16, _acc16, 0)
        return carry
    lax.fori_loop(0, NCHUNK, _acc, 0)

    pltpu.sync_copy(hist_v, out_hbm.at[wid])


@functools.cache
def _deg_kernel():
    return pl.kernel(
        _deg_body,
        out_type=jax.ShapeDtypeStruct((NW, NPAD), jnp.float32),
        mesh=_mesh(),
        scratch_types=[
            pltpu.VMEM((NCHUNK, CH), jnp.int32),
            pltpu.VMEM((NPAD,), jnp.float32),
        ],
        compiler_params=pltpu.CompilerParams(needs_layout_passes=False),
    )


def _mp_body(y_hbm, srcv_hbm, dstv_hbm, zrows_hbm, out_hbm,
             iw0, iw1, iw2, buf0, buf1, buf2, acc_sh,
             rs0, rs1, rs2, ss0, ss1, ss2, is0, is1, is2):
    c = lax.axis_index("c")
    s = lax.axis_index("s")
    wid = s * NC + c
    iws = (iw0, iw1, iw2)
    bufs = (buf0, buf1, buf2)
    rss = (rs0, rs1, rs2)
    sss = (ss0, ss1, ss2)
    iss = (is0, is1, is2)

    def _fetch_idx(t, j):
        pltpu.async_copy(srcv_hbm.at[wid, j], iws[t].at[0], iss[t])
        pltpu.async_copy(dstv_hbm.at[wid, j], iws[t].at[1], iss[t])
        pltpu.make_async_copy(srcv_hbm.at[wid, j], iws[t].at[0], iss[t]).wait()
        pltpu.make_async_copy(dstv_hbm.at[wid, j], iws[t].at[1], iss[t]).wait()

    pltpu.sync_copy(zrows_hbm, acc_sh.at[pl.ds(s * SUBROWS, SUBROWS)])
    for t in range(3):
        _fetch_idx(t, t)
        pltpu.async_copy(y_hbm.at[iws[t].at[0]], bufs[t], rss[t])
    plsc.subcore_barrier()

    def _triplet(g, carry):
        j0 = 3 * g
        for t in range(3):
            pltpu.make_async_copy(y_hbm.at[iws[t].at[0]], bufs[t],
                                  rss[t]).wait()
            pltpu.async_copy(bufs[t], acc_sh.at[iws[t].at[1]], sss[t],
                             add=True)
        @pl.when(g < NCHUNK // 3 - 1)
        def _():
            for t in range(3):
                pltpu.make_async_copy(bufs[t], acc_sh.at[iws[t].at[1]],
                                      sss[t]).wait()
                _fetch_idx(t, j0 + 3 + t)
                pltpu.async_copy(y_hbm.at[iws[t].at[0]], bufs[t], rss[t])
        return carry
    lax.fori_loop(0, NCHUNK // 3, _triplet, 0)

    for t in range(3):
        pltpu.make_async_copy(bufs[t], acc_sh.at[iws[t].at[1]],
                              sss[t]).wait()
    plsc.subcore_barrier()
    pltpu.sync_copy(acc_sh.at[pl.ds(s * SUBROWS, SUBROWS)],
                    out_hbm.at[c, pl.ds(s * SUBROWS, SUBROWS)])


@functools.cache
def _mp_kernel():
    return pl.kernel(
        _mp_body,
        out_type=jax.ShapeDtypeStruct((NC, NPAD, F), jnp.float32),
        mesh=_mesh(),
        scratch_types=[
            pltpu.VMEM((2, CH), jnp.int32),
            pltpu.VMEM((2, CH), jnp.int32),
            pltpu.VMEM((2, CH), jnp.int32),
            pltpu.VMEM((CH, F), jnp.float32),
            pltpu.VMEM((CH, F), jnp.float32),
            pltpu.VMEM((CH, F), jnp.float32),
            pltpu.VMEM_SHARED((NPAD, F), jnp.float32),
            pltpu.SemaphoreType.DMA,
            pltpu.SemaphoreType.DMA,
            pltpu.SemaphoreType.DMA,
            pltpu.SemaphoreType.DMA,
            pltpu.SemaphoreType.DMA,
            pltpu.SemaphoreType.DMA,
            pltpu.SemaphoreType.DMA,
            pltpu.SemaphoreType.DMA,
            pltpu.SemaphoreType.DMA,
        ],
        compiler_params=pltpu.CompilerParams(needs_layout_passes=False),
    )


def _prep_body(degp_ref, dinv_ref):
    deg = jnp.sum(degp_ref[...], axis=0, keepdims=True) + 1.0
    r = lax.rsqrt(deg)
    dinv_ref[...] = r * (1.5 - 0.5 * deg * r * r)


def _l1_body(x_ref, W_ref, dinv_ref, y_ref):
    xw = jnp.dot(x_ref[...], W_ref[...],
                 preferred_element_type=jnp.float32)
    y_ref[...] = jnp.concatenate(
        [xw * dinv_ref[0:N], jnp.zeros((NPAD - N, F), jnp.float32)], axis=0)


def _layer_body(acc_ref, yprev_ref, dinv_ref, b_ref, W_ref, y_ref):
    pre = (acc_ref[0] + acc_ref[1] + yprev_ref[...]) * dinv_ref[...] + b_ref[...]
    h = jnp.maximum(pre, 0.0)
    y_ref[...] = jnp.dot(h, W_ref[...],
                         preferred_element_type=jnp.float32) * dinv_ref[...]


def _final_body(acc_ref, y3_ref, dinv_ref, b3_ref, batch_ref,
                Wl1_ref, bl1_ref, Wl2_ref, bl2_ref, out_ref):
    out3 = (acc_ref[0] + acc_ref[1] + y3_ref[...]) * dinv_ref[...] + b3_ref[...]
    gid = lax.broadcasted_iota(jnp.int32, (G, NPAD), 0)
    maskT = (batch_ref[...] == gid).astype(jnp.float32)
    psum = jnp.dot(maskT, out3,
                   preferred_element_type=jnp.float32, precision=_HIGH)
    counts = jnp.sum(maskT, axis=1, keepdims=True)
    pooled = psum / jnp.maximum(counts, 1.0)
    z = jnp.maximum(
        jnp.dot(pooled, Wl1_ref[...],
                preferred_element_type=jnp.float32)
        + bl1_ref[...], 0.0)
    out_ref[...] = jnp.dot(z, Wl2_ref[...],
                           preferred_element_type=jnp.float32) + bl2_ref[...]


def _tc_call(body, out_shape):
    return pl.pallas_call(body, out_shape=out_shape)


def kernel(x, edge_index, batch, W1, b1, W2, b2, W3, b3, Wl1, bl1, Wl2, bl2):
    f32 = jnp.float32
    pads = jnp.arange(EP - E, dtype=jnp.int32)
    src = jnp.concatenate([edge_index[0], pads % N])
    dst = jnp.concatenate([edge_index[1], N + pads % (NPAD - N)])
    srcv = src.reshape(NW, NCHUNK, CH)
    dstv = dst.reshape(NW, NCHUNK, CH)
    batch_row = jnp.concatenate([batch,
                                 jnp.full((NPAD - N,), G, jnp.int32)]
                                ).reshape(1, NPAD)
    zrows = jnp.zeros((SUBROWS, F), f32)

    deg_part = _deg_kernel()(dstv)
    dinv_row = _tc_call(_prep_body,
                        jax.ShapeDtypeStruct((1, NPAD), f32))(deg_part)
    dinv = dinv_row.reshape(NPAD, 1)

    y_shape = jax.ShapeDtypeStruct((NPAD, F), f32)
    y1 = _tc_call(_l1_body, y_shape)(x, W1, dinv)
    acc1 = _mp_kernel()(y1, srcv, dstv, zrows)
    y2 = _tc_call(_layer_body, y_shape)(acc1, y1, dinv, b1.reshape(1, F), W2)
    acc2 = _mp_kernel()(y2, srcv, dstv, zrows)
    y3 = _tc_call(_layer_body, y_shape)(acc2, y2, dinv, b2.reshape(1, F), W3)
    acc3 = _mp_kernel()(y3, srcv, dstv, zrows)

    out = _tc_call(_final_body, jax.ShapeDtypeStruct((G, C), f32))(
        acc3, y3, dinv, b3.reshape(1, F), batch_row,
        Wl1, bl1.reshape(1, F), Wl2, bl2.reshape(1, C))
    return out


# speedup vs baseline: 28.0775x; 1.0208x over previous
import functools

import jax
import jax.numpy as jnp
from jax import lax
from jax.experimental import pallas as pl
from jax.experimental.pallas import tpu as pltpu
from jax.experimental.pallas import tpu_sc as plsc

N = 10000
E = 320000
F = 128
C = 10
G = 64

NC, NS = 2, 16
NW = NC * NS
NPAD = 10112
SUBROWS = NPAD // NS
CH = 128
NCHUNK = 81
EPW = NCHUNK * CH
EP = NW * EPW

_HIGH = jax.lax.Precision.HIGHEST


@functools.cache
def _mesh():
    return plsc.VectorSubcoreMesh(core_axis_name="c", subcore_axis_name="s",
                                  num_cores=NC, num_subcores=NS)


def _deg_body(dstv_hbm, out_hbm, dst_v, hist_v):
    c = lax.axis_index("c")
    s = lax.axis_index("s")
    wid = s * NC + c
    pltpu.sync_copy(dstv_hbm.at[wid], dst_v)

    zeros16 = jnp.zeros((16,), jnp.float32)
    def _zero(i, carry):
        hist_v[pl.ds(i * 16, 16)] = zeros16
        return carry
    lax.fori_loop(0, NPAD // 16, _zero, 0)

    ones16 = jnp.ones((16,), jnp.float32)
    def _acc(j, carry):
        def _acc16(k, carry2):
            idx = dst_v[j, pl.ds(k * 16, 16)]
            plsc.addupdate_scatter(hist_v, [idx], ones16)
            return carry2
        lax.fori_loop(0, CH // 16, _acc16, 0)
        return carry
    lax.fori_loop(0, NCHUNK, _acc, 0)

    pltpu.sync_copy(hist_v, out_hbm.at[wid])


@functools.cache
def _deg_kernel():
    return pl.kernel(
        _deg_body,
        out_type=jax.ShapeDtypeStruct((NW, NPAD), jnp.float32),
        mesh=_mesh(),
        scratch_types=[
            pltpu.VMEM((NCHUNK, CH), jnp.int32),
            pltpu.VMEM((NPAD,), jnp.float32),
        ],
        compiler_params=pltpu.CompilerParams(needs_layout_passes=False),
    )


def _mp_body(y_hbm, srcv_hbm, dstv_hbm, zrows_hbm, out_hbm,
             iw0, iw1, iw2, buf0, buf1, buf2, acc_sh,
             rs0, rs1, rs2, ss0, ss1, ss2, is0, is1, is2):
    c = lax.axis_index("c")
    s = lax.axis_index("s")
    wid = s * NC + c
    iws = (iw0, iw1, iw2)
    bufs = (buf0, buf1, buf2)
    rss = (rs0, rs1, rs2)
    sss = (ss0, ss1, ss2)
    iss = (is0, is1, is2)

    def _fetch_idx(t, j):
        pltpu.async_copy(srcv_hbm.at[wid, j], iws[t].at[0], iss[t])
        pltpu.async_copy(dstv_hbm.at[wid, j], iws[t].at[1], iss[t])
        pltpu.make_async_copy(srcv_hbm.at[wid, j], iws[t].at[0], iss[t]).wait()
        pltpu.make_async_copy(dstv_hbm.at[wid, j], iws[t].at[1], iss[t]).wait()

    pltpu.sync_copy(zrows_hbm, acc_sh.at[pl.ds(s * SUBROWS, SUBROWS)])
    for t in range(3):
        _fetch_idx(t, t)
        pltpu.async_copy(y_hbm.at[iws[t].at[0]], bufs[t], rss[t])
    plsc.subcore_barrier()

    def _triplet(g, carry):
        j0 = 3 * g
        for t in range(3):
            pltpu.make_async_copy(y_hbm.at[iws[t].at[0]], bufs[t],
                                  rss[t]).wait()
            pltpu.async_copy(bufs[t], acc_sh.at[iws[t].at[1]], sss[t],
                             add=True)
        @pl.when(g < NCHUNK // 3 - 1)
        def _():
            for t in range(3):
                pltpu.make_async_copy(bufs[t], acc_sh.at[iws[t].at[1]],
                                      sss[t]).wait()
                _fetch_idx(t, j0 + 3 + t)
                pltpu.async_copy(y_hbm.at[iws[t].at[0]], bufs[t], rss[t])
        return carry
    lax.fori_loop(0, NCHUNK // 3, _triplet, 0)

    for t in range(3):
        pltpu.make_async_copy(bufs[t], acc_sh.at[iws[t].at[1]],
                              sss[t]).wait()
    plsc.subcore_barrier()
    pltpu.sync_copy(acc_sh.at[pl.ds(s * SUBROWS, SUBROWS)],
                    out_hbm.at[c, pl.ds(s * SUBROWS, SUBROWS)])


@functools.cache
def _mp_kernel():
    return pl.kernel(
        _mp_body,
        out_type=jax.ShapeDtypeStruct((NC, NPAD, F), jnp.float32),
        mesh=_mesh(),
        scratch_types=[
            pltpu.VMEM((2, CH), jnp.int32),
            pltpu.VMEM((2, CH), jnp.int32),
            pltpu.VMEM((2, CH), jnp.int32),
            pltpu.VMEM((CH, F), jnp.float32),
            pltpu.VMEM((CH, F), jnp.float32),
            pltpu.VMEM((CH, F), jnp.float32),
            pltpu.VMEM_SHARED((NPAD, F), jnp.float32),
            pltpu.SemaphoreType.DMA,
            pltpu.SemaphoreType.DMA,
            pltpu.SemaphoreType.DMA,
            pltpu.SemaphoreType.DMA,
            pltpu.SemaphoreType.DMA,
            pltpu.SemaphoreType.DMA,
            pltpu.SemaphoreType.DMA,
            pltpu.SemaphoreType.DMA,
            pltpu.SemaphoreType.DMA,
        ],
        compiler_params=pltpu.CompilerParams(needs_layout_passes=False),
    )


def _split_body(ei_ref, src_ref, dst_ref):
    pads = lax.broadcasted_iota(jnp.int32, (1, EP - E), 1)
    src_ref[...] = jnp.concatenate(
        [ei_ref[0, :], (pads % N).reshape(EP - E)])
    dst_ref[...] = jnp.concatenate(
        [ei_ref[1, :], (N + pads % (NPAD - N)).reshape(EP - E)])


def _prep_body(degp_ref, dinv_ref):
    deg = jnp.sum(degp_ref[...], axis=0, keepdims=True) + 1.0
    r = lax.rsqrt(deg)
    dinv_ref[...] = r * (1.5 - 0.5 * deg * r * r)


def _l1_body(x_ref, W_ref, dinv_ref, y_ref):
    xw = jnp.dot(x_ref[...], W_ref[...],
                 preferred_element_type=jnp.float32)
    y_ref[...] = jnp.concatenate(
        [xw * dinv_ref[0:N], jnp.zeros((NPAD - N, F), jnp.float32)], axis=0)


def _layer_body(acc_ref, yprev_ref, dinv_ref, b_ref, W_ref, y_ref):
    pre = (acc_ref[0] + acc_ref[1] + yprev_ref[...]) * dinv_ref[...] + b_ref[...]
    h = jnp.maximum(pre, 0.0)
    y_ref[...] = jnp.dot(h, W_ref[...],
                         preferred_element_type=jnp.float32) * dinv_ref[...]


def _final_body(acc_ref, y3_ref, dinv_ref, b3_ref, batch_ref,
                Wl1_ref, bl1_ref, Wl2_ref, bl2_ref, out_ref):
    out3 = (acc_ref[0] + acc_ref[1] + y3_ref[...]) * dinv_ref[...] + b3_ref[...]
    gid = lax.broadcasted_iota(jnp.int32, (G, NPAD), 0)
    maskT = (batch_ref[...] == gid).astype(jnp.float32)
    psum = jnp.dot(maskT, out3,
                   preferred_element_type=jnp.float32, precision=_HIGH)
    counts = jnp.sum(maskT, axis=1, keepdims=True)
    pooled = psum / jnp.maximum(counts, 1.0)
    z = jnp.maximum(
        jnp.dot(pooled, Wl1_ref[...],
                preferred_element_type=jnp.float32)
        + bl1_ref[...], 0.0)
    out_ref[...] = jnp.dot(z, Wl2_ref[...],
                           preferred_element_type=jnp.float32) + bl2_ref[...]


def _tc_call(body, out_shape):
    return pl.pallas_call(body, out_shape=out_shape)


def kernel(x, edge_index, batch, W1, b1, W2, b2, W3, b3, Wl1, bl1, Wl2, bl2):
    f32 = jnp.float32
    src, dst = pl.pallas_call(
        _split_body,
        out_shape=[jax.ShapeDtypeStruct((EP,), jnp.int32),
                   jax.ShapeDtypeStruct((EP,), jnp.int32)],
    )(edge_index)
    srcv = src.reshape(NW, NCHUNK, CH)
    dstv = dst.reshape(NW, NCHUNK, CH)
    batch_row = jnp.concatenate([batch,
                                 jnp.full((NPAD - N,), G, jnp.int32)]
                                ).reshape(1, NPAD)
    zrows = jnp.zeros((SUBROWS, F), f32)

    deg_part = _deg_kernel()(dstv)
    dinv_row = _tc_call(_prep_body,
                        jax.ShapeDtypeStruct((1, NPAD), f32))(deg_part)
    dinv = dinv_row.reshape(NPAD, 1)

    y_shape = jax.ShapeDtypeStruct((NPAD, F), f32)
    y1 = _tc_call(_l1_body, y_shape)(x, W1, dinv)
    acc1 = _mp_kernel()(y1, srcv, dstv, zrows)
    y2 = _tc_call(_layer_body, y_shape)(acc1, y1, dinv, b1.reshape(1, F), W2)
    acc2 = _mp_kernel()(y2, srcv, dstv, zrows)
    y3 = _tc_call(_layer_body, y_shape)(acc2, y2, dinv, b2.reshape(1, F), W3)
    acc3 = _mp_kernel()(y3, srcv, dstv, zrows)

    out = _tc_call(_final_body, jax.ShapeDtypeStruct((G, C), f32))(
        acc3, y3, dinv, b3.reshape(1, F), batch_row,
        Wl1, bl1.reshape(1, F), Wl2, bl2.reshape(1, C))
    return out


# speedup vs baseline: 28.1027x; 1.0009x over previous
import functools

import jax
import jax.numpy as jnp
from jax import lax
from jax.experimental import pallas as pl
from jax.experimental.pallas import tpu as pltpu
from jax.experimental.pallas import tpu_sc as plsc

N = 10000
E = 320000
F = 128
C = 10
G = 64

NC, NS = 2, 16
NW = NC * NS
NPAD = 10112
SUBROWS = NPAD // NS
CH = 128
NCHUNK = 81
EPW = NCHUNK * CH
EP = NW * EPW

_HIGH = jax.lax.Precision.HIGHEST


@functools.cache
def _mesh():
    return plsc.VectorSubcoreMesh(core_axis_name="c", subcore_axis_name="s",
                                  num_cores=NC, num_subcores=NS)


def _deg_body(dstv_hbm, out_hbm, dst_v, hist_v):
    c = lax.axis_index("c")
    s = lax.axis_index("s")
    wid = s * NC + c
    pltpu.sync_copy(dstv_hbm.at[wid], dst_v)

    zeros16 = jnp.zeros((16,), jnp.float32)
    def _zero(i, carry):
        hist_v[pl.ds(i * 16, 16)] = zeros16
        return carry
    lax.fori_loop(0, NPAD // 16, _zero, 0)

    ones16 = jnp.ones((16,), jnp.float32)
    def _acc(j, carry):
        def _acc16(k, carry2):
            idx = dst_v[j, pl.ds(k * 16, 16)]
            plsc.addupdate_scatter(hist_v, [idx], ones16)
            return carry2
        lax.fori_loop(0, CH // 16, _acc16, 0)
        return carry
    lax.fori_loop(0, NCHUNK, _acc, 0)

    pltpu.sync_copy(hist_v, out_hbm.at[wid])


@functools.cache
def _deg_kernel():
    return pl.kernel(
        _deg_body,
        out_type=jax.ShapeDtypeStruct((NW, NPAD), jnp.float32),
        mesh=_mesh(),
        scratch_types=[
            pltpu.VMEM((NCHUNK, CH), jnp.int32),
            pltpu.VMEM((NPAD,), jnp.float32),
        ],
        compiler_params=pltpu.CompilerParams(needs_layout_passes=False),
    )


def _mp_body(y_hbm, srcv_hbm, dstv_hbm, zrows_hbm, out_hbm,
             iw0, iw1, iw2, buf0, buf1, buf2, acc_sh,
             rs0, rs1, rs2, ss0, ss1, ss2, is0, is1, is2):
    c = lax.axis_index("c")
    s = lax.axis_index("s")
    wid = s * NC + c
    iws = (iw0, iw1, iw2)
    bufs = (buf0, buf1, buf2)
    rss = (rs0, rs1, rs2)
    sss = (ss0, ss1, ss2)
    iss = (is0, is1, is2)

    def _fetch_idx(t, j):
        pltpu.async_copy(srcv_hbm.at[wid, j], iws[t].at[0], iss[t])
        pltpu.async_copy(dstv_hbm.at[wid, j], iws[t].at[1], iss[t])
        pltpu.make_async_copy(srcv_hbm.at[wid, j], iws[t].at[0], iss[t]).wait()
        pltpu.make_async_copy(dstv_hbm.at[wid, j], iws[t].at[1], iss[t]).wait()

    pltpu.sync_copy(zrows_hbm, acc_sh.at[pl.ds(s * SUBROWS, SUBROWS)])
    for t in range(3):
        _fetch_idx(t, t)
        pltpu.async_copy(y_hbm.at[iws[t].at[0]], bufs[t], rss[t])
    plsc.subcore_barrier()

    def _triplet(g, carry):
        j0 = 3 * g
        for t in range(3):
            pltpu.make_async_copy(y_hbm.at[iws[t].at[0]], bufs[t],
                                  rss[t]).wait()
            pltpu.async_copy(bufs[t], acc_sh.at[iws[t].at[1]], sss[t],
                             add=True)
        @pl.when(g < NCHUNK // 3 - 1)
        def _():
            for t in range(3):
                pltpu.make_async_copy(bufs[t], acc_sh.at[iws[t].at[1]],
                                      sss[t]).wait()
                _fetch_idx(t, j0 + 3 + t)
                pltpu.async_copy(y_hbm.at[iws[t].at[0]], bufs[t], rss[t])
        return carry
    lax.fori_loop(0, NCHUNK // 3, _triplet, 0)

    for t in range(3):
        pltpu.make_async_copy(bufs[t], acc_sh.at[iws[t].at[1]],
                              sss[t]).wait()
    plsc.subcore_barrier()
    pltpu.sync_copy(acc_sh.at[pl.ds(s * SUBROWS, SUBROWS)],
                    out_hbm.at[c, pl.ds(s * SUBROWS, SUBROWS)])


@functools.cache
def _mp_kernel():
    return pl.kernel(
        _mp_body,
        out_type=jax.ShapeDtypeStruct((NC, NPAD, F), jnp.float32),
        mesh=_mesh(),
        scratch_types=[
            pltpu.VMEM((2, CH), jnp.int32),
            pltpu.VMEM((2, CH), jnp.int32),
            pltpu.VMEM((2, CH), jnp.int32),
            pltpu.VMEM((CH, F), jnp.float32),
            pltpu.VMEM((CH, F), jnp.float32),
            pltpu.VMEM((CH, F), jnp.float32),
            pltpu.VMEM_SHARED((NPAD, F), jnp.float32),
            pltpu.SemaphoreType.DMA,
            pltpu.SemaphoreType.DMA,
            pltpu.SemaphoreType.DMA,
            pltpu.SemaphoreType.DMA,
            pltpu.SemaphoreType.DMA,
            pltpu.SemaphoreType.DMA,
            pltpu.SemaphoreType.DMA,
            pltpu.SemaphoreType.DMA,
            pltpu.SemaphoreType.DMA,
        ],
        compiler_params=pltpu.CompilerParams(needs_layout_passes=False),
    )


def _split_body(ei_ref, src_ref, dst_ref):
    pads = lax.broadcasted_iota(jnp.int32, (1, EP - E), 1)
    src_ref[...] = jnp.concatenate(
        [ei_ref[0, :], (pads % N).reshape(EP - E)])
    dst_ref[...] = jnp.concatenate(
        [ei_ref[1, :], (N + pads % (NPAD - N)).reshape(EP - E)])


def _prep_body(degp_ref, dinv_ref):
    deg = jnp.sum(degp_ref[...], axis=0, keepdims=True) + 1.0
    r = lax.rsqrt(deg)
    dinv_ref[...] = (r * (1.5 - 0.5 * deg * r * r)).reshape(NPAD, 1)


def _l1_body(x_ref, W_ref, dinv_ref, y_ref):
    xw = jnp.dot(x_ref[...], W_ref[...],
                 preferred_element_type=jnp.float32)
    y_ref[...] = jnp.concatenate(
        [xw * dinv_ref[0:N], jnp.zeros((NPAD - N, F), jnp.float32)], axis=0)


def _layer_body(acc_ref, yprev_ref, dinv_ref, b_ref, W_ref, y_ref):
    pre = (acc_ref[0] + acc_ref[1] + yprev_ref[...]) * dinv_ref[...] + b_ref[...]
    h = jnp.maximum(pre, 0.0)
    y_ref[...] = jnp.dot(h, W_ref[...],
                         preferred_element_type=jnp.float32) * dinv_ref[...]


def _final_body(acc_ref, y3_ref, dinv_ref, b3_ref, batch_ref,
                Wl1_ref, bl1_ref, Wl2_ref, bl2_ref, out_ref):
    out3 = (acc_ref[0] + acc_ref[1] + y3_ref[...]) * dinv_ref[...] + b3_ref[...]
    gid = lax.broadcasted_iota(jnp.int32, (G, NPAD), 0)
    maskT = (batch_ref[...] == gid).astype(jnp.float32)
    psum = jnp.dot(maskT, out3,
                   preferred_element_type=jnp.float32, precision=_HIGH)
    counts = jnp.sum(maskT, axis=1, keepdims=True)
    pooled = psum / jnp.maximum(counts, 1.0)
    z = jnp.maximum(
        jnp.dot(pooled, Wl1_ref[...],
                preferred_element_type=jnp.float32)
        + bl1_ref[...], 0.0)
    out_ref[...] = jnp.dot(z, Wl2_ref[...],
                           preferred_element_type=jnp.float32) + bl2_ref[...]


def _tc_call(body, out_shape):
    return pl.pallas_call(body, out_shape=out_shape)


def kernel(x, edge_index, batch, W1, b1, W2, b2, W3, b3, Wl1, bl1, Wl2, bl2):
    f32 = jnp.float32
    src, dst = pl.pallas_call(
        _split_body,
        out_shape=[jax.ShapeDtypeStruct((EP,), jnp.int32),
                   jax.ShapeDtypeStruct((EP,), jnp.int32)],
    )(edge_index)
    srcv = src.reshape(NW, NCHUNK, CH)
    dstv = dst.reshape(NW, NCHUNK, CH)
    batch_row = jnp.concatenate([batch,
                                 jnp.full((NPAD - N,), G, jnp.int32)]
                                ).reshape(1, NPAD)
    zrows = jnp.zeros((SUBROWS, F), f32)

    deg_part = _deg_kernel()(dstv)
    dinv = _tc_call(_prep_body,
                    jax.ShapeDtypeStruct((NPAD, 1), f32))(deg_part)

    y_shape = jax.ShapeDtypeStruct((NPAD, F), f32)
    y1 = _tc_call(_l1_body, y_shape)(x, W1, dinv)
    acc1 = _mp_kernel()(y1, srcv, dstv, zrows)
    y2 = _tc_call(_layer_body, y_shape)(acc1, y1, dinv, b1.reshape(1, F), W2)
    acc2 = _mp_kernel()(y2, srcv, dstv, zrows)
    y3 = _tc_call(_layer_body, y_shape)(acc2, y2, dinv, b2.reshape(1, F), W3)
    acc3 = _mp_kernel()(y3, srcv, dstv, zrows)

    out = _tc_call(_final_body, jax.ShapeDtypeStruct((G, C), f32))(
        acc3, y3, dinv, b3.reshape(1, F), batch_row,
        Wl1, bl1.reshape(1, F), Wl2, bl2.reshape(1, C))
    return out


# speedup vs baseline: 28.2128x; 1.0039x over previous
import functools

import jax
import jax.numpy as jnp
from jax import lax
from jax.experimental import pallas as pl
from jax.experimental.pallas import tpu as pltpu
from jax.experimental.pallas import tpu_sc as plsc

N = 10000
E = 320000
F = 128
C = 10
G = 64

NC, NS = 2, 16
NW = NC * NS
NPAD = 10112
SUBROWS = NPAD // NS
CH = 128
NCHUNK = 81
EPW = NCHUNK * CH
EP = NW * EPW

_HIGH = jax.lax.Precision.HIGHEST


@functools.cache
def _mesh():
    return plsc.VectorSubcoreMesh(core_axis_name="c", subcore_axis_name="s",
                                  num_cores=NC, num_subcores=NS)


def _deg_body(dstv_hbm, out_hbm, dst_v, hist_v):
    c = lax.axis_index("c")
    s = lax.axis_index("s")
    wid = s * NC + c
    pltpu.sync_copy(dstv_hbm.at[wid], dst_v)

    zeros16 = jnp.zeros((16,), jnp.float32)
    def _zero(i, carry):
        hist_v[pl.ds(i * 16, 16)] = zeros16
        return carry
    lax.fori_loop(0, NPAD // 16, _zero, 0)

    ones16 = jnp.ones((16,), jnp.float32)
    def _acc(j, carry):
        def _acc16(k, carry2):
            idx = dst_v[j, pl.ds(k * 16, 16)]
            plsc.addupdate_scatter(hist_v, [idx], ones16)
            return carry2
        lax.fori_loop(0, CH // 16, _acc16, 0)
        return carry
    lax.fori_loop(0, NCHUNK, _acc, 0)

    pltpu.sync_copy(hist_v, out_hbm.at[wid])


@functools.cache
def _deg_kernel():
    return pl.kernel(
        _deg_body,
        out_type=jax.ShapeDtypeStruct((NW, NPAD), jnp.float32),
        mesh=_mesh(),
        scratch_types=[
            pltpu.VMEM((NCHUNK, CH), jnp.int32),
            pltpu.VMEM((NPAD,), jnp.float32),
        ],
        compiler_params=pltpu.CompilerParams(needs_layout_passes=False),
    )


def _mp_body(y_hbm, srcv_hbm, dstv_hbm, zrows_hbm, out_hbm,
             iw0, iw1, iw2, buf0, buf1, buf2, acc_sh,
             rs0, rs1, rs2, ss0, ss1, ss2, is0, is1, is2):
    c = lax.axis_index("c")
    s = lax.axis_index("s")
    wid = s * NC + c
    iws = (iw0, iw1, iw2)
    bufs = (buf0, buf1, buf2)
    rss = (rs0, rs1, rs2)
    sss = (ss0, ss1, ss2)
    iss = (is0, is1, is2)

    def _fetch_idx(t, j):
        pltpu.async_copy(srcv_hbm.at[wid, j], iws[t].at[0], iss[t])
        pltpu.async_copy(dstv_hbm.at[wid, j], iws[t].at[1], iss[t])
        pltpu.make_async_copy(srcv_hbm.at[wid, j], iws[t].at[0], iss[t]).wait()
        pltpu.make_async_copy(dstv_hbm.at[wid, j], iws[t].at[1], iss[t]).wait()

    for t in range(3):
        _fetch_idx(t, t)
        pltpu.async_copy(y_hbm.at[iws[t].at[0]], bufs[t], rss[t])
    pltpu.sync_copy(zrows_hbm, acc_sh.at[pl.ds(s * SUBROWS, SUBROWS)])
    plsc.subcore_barrier()

    def _triplet(g, carry):
        j0 = 3 * g
        for t in range(3):
            pltpu.make_async_copy(y_hbm.at[iws[t].at[0]], bufs[t],
                                  rss[t]).wait()
            pltpu.async_copy(bufs[t], acc_sh.at[iws[t].at[1]], sss[t],
                             add=True)
        @pl.when(g < NCHUNK // 3 - 1)
        def _():
            for t in range(3):
                pltpu.make_async_copy(bufs[t], acc_sh.at[iws[t].at[1]],
                                      sss[t]).wait()
                _fetch_idx(t, j0 + 3 + t)
                pltpu.async_copy(y_hbm.at[iws[t].at[0]], bufs[t], rss[t])
        return carry
    lax.fori_loop(0, NCHUNK // 3, _triplet, 0)

    for t in range(3):
        pltpu.make_async_copy(bufs[t], acc_sh.at[iws[t].at[1]],
                              sss[t]).wait()
    plsc.subcore_barrier()
    pltpu.sync_copy(acc_sh.at[pl.ds(s * SUBROWS, SUBROWS)],
                    out_hbm.at[c, pl.ds(s * SUBROWS, SUBROWS)])


@functools.cache
def _mp_kernel():
    return pl.kernel(
        _mp_body,
        out_type=jax.ShapeDtypeStruct((NC, NPAD, F), jnp.float32),
        mesh=_mesh(),
        scratch_types=[
            pltpu.VMEM((2, CH), jnp.int32),
            pltpu.VMEM((2, CH), jnp.int32),
            pltpu.VMEM((2, CH), jnp.int32),
            pltpu.VMEM((CH, F), jnp.float32),
            pltpu.VMEM((CH, F), jnp.float32),
            pltpu.VMEM((CH, F), jnp.float32),
            pltpu.VMEM_SHARED((NPAD, F), jnp.float32),
            pltpu.SemaphoreType.DMA,
            pltpu.SemaphoreType.DMA,
            pltpu.SemaphoreType.DMA,
            pltpu.SemaphoreType.DMA,
            pltpu.SemaphoreType.DMA,
            pltpu.SemaphoreType.DMA,
            pltpu.SemaphoreType.DMA,
            pltpu.SemaphoreType.DMA,
            pltpu.SemaphoreType.DMA,
        ],
        compiler_params=pltpu.CompilerParams(needs_layout_passes=False),
    )


def _split_body(ei_ref, src_ref, dst_ref):
    pads = lax.broadcasted_iota(jnp.int32, (1, EP - E), 1)
    src_ref[...] = jnp.concatenate(
        [ei_ref[0, :], (pads % N).reshape(EP - E)])
    dst_ref[...] = jnp.concatenate(
        [ei_ref[1, :], (N + pads % (NPAD - N)).reshape(EP - E)])


def _prep_body(degp_ref, dinv_ref):
    deg = jnp.sum(degp_ref[...], axis=0, keepdims=True) + 1.0
    r = lax.rsqrt(deg)
    dinv_ref[...] = (r * (1.5 - 0.5 * deg * r * r)).reshape(NPAD, 1)


def _l1_body(x_ref, W_ref, dinv_ref, y_ref):
    xw = jnp.dot(x_ref[...], W_ref[...],
                 preferred_element_type=jnp.float32)
    y_ref[...] = jnp.concatenate(
        [xw * dinv_ref[0:N], jnp.zeros((NPAD - N, F), jnp.float32)], axis=0)


def _layer_body(acc_ref, yprev_ref, dinv_ref, b_ref, W_ref, y_ref):
    pre = (acc_ref[0] + acc_ref[1] + yprev_ref[...]) * dinv_ref[...] + b_ref[...]
    h = jnp.maximum(pre, 0.0)
    y_ref[...] = jnp.dot(h, W_ref[...],
                         preferred_element_type=jnp.float32) * dinv_ref[...]


def _final_body(acc_ref, y3_ref, dinv_ref, b3_ref, batch_ref,
                Wl1_ref, bl1_ref, Wl2_ref, bl2_ref, out_ref):
    out3 = (acc_ref[0] + acc_ref[1] + y3_ref[...]) * dinv_ref[...] + b3_ref[...]
    gid = lax.broadcasted_iota(jnp.int32, (G, NPAD), 0)
    maskT = (batch_ref[...] == gid).astype(jnp.float32)
    psum = jnp.dot(maskT, out3,
                   preferred_element_type=jnp.float32, precision=_HIGH)
    counts = jnp.sum(maskT, axis=1, keepdims=True)
    pooled = psum / jnp.maximum(counts, 1.0)
    z = jnp.maximum(
        jnp.dot(pooled, Wl1_ref[...],
                preferred_element_type=jnp.float32)
        + bl1_ref[...], 0.0)
    out_ref[...] = jnp.dot(z, Wl2_ref[...],
                           preferred_element_type=jnp.float32) + bl2_ref[...]


def _tc_call(body, out_shape):
    return pl.pallas_call(body, out_shape=out_shape)


def kernel(x, edge_index, batch, W1, b1, W2, b2, W3, b3, Wl1, bl1, Wl2, bl2):
    f32 = jnp.float32
    src, dst = pl.pallas_call(
        _split_body,
        out_shape=[jax.ShapeDtypeStruct((EP,), jnp.int32),
                   jax.ShapeDtypeStruct((EP,), jnp.int32)],
    )(edge_index)
    srcv = src.reshape(NW, NCHUNK, CH)
    dstv = dst.reshape(NW, NCHUNK, CH)
    batch_row = jnp.concatenate([batch,
                                 jnp.full((NPAD - N,), G, jnp.int32)]
                                ).reshape(1, NPAD)
    zrows = jnp.zeros((SUBROWS, F), f32)

    deg_part = _deg_kernel()(dstv)
    dinv = _tc_call(_prep_body,
                    jax.ShapeDtypeStruct((NPAD, 1), f32))(deg_part)

    y_shape = jax.ShapeDtypeStruct((NPAD, F), f32)
    y1 = _tc_call(_l1_body, y_shape)(x, W1, dinv)
    acc1 = _mp_kernel()(y1, srcv, dstv, zrows)
    y2 = _tc_call(_layer_body, y_shape)(acc1, y1, dinv, b1.reshape(1, F), W2)
    acc2 = _mp_kernel()(y2, srcv, dstv, zrows)
    y3 = _tc_call(_layer_body, y_shape)(acc2, y2, dinv, b2.reshape(1, F), W3)
    acc3 = _mp_kernel()(y3, srcv, dstv, zrows)

    out = _tc_call(_final_body, jax.ShapeDtypeStruct((G, C), f32))(
        acc3, y3, dinv, b3.reshape(1, F), batch_row,
        Wl1, bl1.reshape(1, F), Wl2, bl2.reshape(1, C))
    return out
